# baseline (speedup 1.0000x reference)
import functools

import jax
import jax.numpy as jnp
from jax import lax
from jax.experimental import pallas as pl
from jax.experimental.pallas import tpu as pltpu
from jax.experimental.pallas import tpu_sc as plsc

NC = 2
NS = 16
NW = NC * NS
CHUNK = 128
NBUF = 1


def _segment_sum_sc(table, src3, dst3):
    n, d = table.shape
    cpw = src3.shape[1] - NBUF
    n_groups = cpw // NBUF
    assert cpw % (2 * NBUF) == 0
    piece = 128
    n_full, tail = divmod(n, piece)
    assert tail % 8 == 0

    mesh = plsc.VectorSubcoreMesh(
        core_axis_name="c", subcore_axis_name="s", num_cores=NC, num_subcores=NS
    )

    @functools.partial(
        pl.kernel,
        out_type=jax.ShapeDtypeStruct((NC, n, d), jnp.float32),
        mesh=mesh,
        scratch_types=[
            [pltpu.VMEM((NBUF, CHUNK), jnp.int32) for _ in range(2)],
            [pltpu.VMEM((NBUF, CHUNK), jnp.int32) for _ in range(2)],
            [pltpu.VMEM((CHUNK, d), jnp.float32) for _ in range(NBUF)],
            pltpu.VMEM_SHARED((n + 128, d), jnp.float32),
            [pltpu.SemaphoreType.DMA for _ in range(NBUF)],
            [pltpu.SemaphoreType.DMA for _ in range(NBUF)],
            [pltpu.SemaphoreType.DMA for _ in range(2)],
        ],
    )
    def segsum(table_hbm, src_hbm, dst_hbm, out_hbm,
               srcb, dstb, rows_l, acc_sh, semg, sems, semi):
        c = lax.axis_index("c")
        s = lax.axis_index("s")
        w = c * NS + s

        def fire_prefetch(j0, slot):
            pltpu.async_copy(src_hbm.at[w].at[pl.ds(j0, NBUF)], srcb[slot], semi[slot])
            pltpu.async_copy(dst_hbm.at[w].at[pl.ds(j0, NBUF)], dstb[slot], semi[slot])

        def drain_prefetch(slot):
            pltpu.make_async_copy(src_hbm.at[w].at[pl.ds(0, NBUF)], srcb[slot],
                                  semi[slot]).wait()
            pltpu.make_async_copy(dst_hbm.at[w].at[pl.ds(0, NBUF)], dstb[slot],
                                  semi[slot]).wait()

        fire_prefetch(0, 0)

        def zbody(i, carry):
            r = i // (d // 16)
            col = (i % (d // 16)) * 16
            rows_l[0][r, pl.ds(col, 16)] = jnp.zeros((16,), jnp.float32)
            return carry

        lax.fori_loop(0, piece * (d // 16), zbody, 0)

        my_pieces = (n_full - 1 - s) // NS + 1

        def zcopy(i, carry):
            r0 = (s + i * NS) * piece
            pltpu.sync_copy(rows_l[0].at[pl.ds(0, piece)], acc_sh.at[pl.ds(r0, piece)])
            return carry

        lax.fori_loop(0, my_pieces, zcopy, 0)
        if tail:
            @pl.when(s == NS - 1)
            def _():
                pltpu.sync_copy(rows_l[0].at[pl.ds(0, tail)],
                                acc_sh.at[pl.ds(n_full * piece, tail)])
        plsc.subcore_barrier()

        def run_group(g, slot):
            fire_prefetch((g + 1) * NBUF, 1 - slot)
            drain_prefetch(slot)
            gd = [pltpu.async_copy(table_hbm.at[srcb[slot].at[b]], rows_l[b], semg[b])
                  for b in range(NBUF)]
            sd = []
            for b in range(NBUF):
                gd[b].wait()
                sd.append(pltpu.async_copy(rows_l[b], acc_sh.at[dstb[slot].at[b]],
                                           sems[b], add=True))
            for b in range(NBUF):
                sd[b].wait()

        def gbody(g2, carry):
            run_group(2 * g2, 0)
            run_group(2 * g2 + 1, 1)
            return carry

        lax.fori_loop(0, n_groups // 2, gbody, 0)
        drain_prefetch(0)
        plsc.subcore_barrier()

        def obody(i, carry):
            r0 = (s + i * NS) * piece
            pltpu.sync_copy(acc_sh.at[pl.ds(r0, piece)], rows_l[0].at[pl.ds(0, piece)])
            pltpu.sync_copy(rows_l[0].at[pl.ds(0, piece)], out_hbm.at[c].at[pl.ds(r0, piece)])
            return carry

        lax.fori_loop(0, my_pieces, obody, 0)
        if tail:
            @pl.when(s == NS - 1)
            def _():
                r0 = n_full * piece
                pltpu.sync_copy(acc_sh.at[pl.ds(r0, tail)], rows_l[0].at[pl.ds(0, tail)])
                pltpu.sync_copy(rows_l[0].at[pl.ds(0, tail)], out_hbm.at[c].at[pl.ds(r0, tail)])

    return segsum(table, src3, dst3)


def _chunk_edges(src, dst, n):
    e = src.shape[0]
    cpw = -(-e // (NW * CHUNK))
    cpw = -(-cpw // (2 * NBUF)) * (2 * NBUF)
    pad = NW * cpw * CHUNK - e
    if pad:
        src = jnp.concatenate([src, jnp.zeros((pad,), jnp.int32)])
        dst = jnp.concatenate(
            [dst, (n + (jnp.arange(pad, dtype=jnp.int32) % 128))])
    over = jnp.zeros((NW, NBUF, CHUNK), jnp.int32)
    src3 = jnp.concatenate([src.reshape(NW, cpw, CHUNK), over], axis=1)
    dst3 = jnp.concatenate([dst.reshape(NW, cpw, CHUNK), over], axis=1)
    return src3, dst3


def _dense_tc(parts, w_mu, w_rho, eps_w, b_mu, b_rho, eps_b, relu):
    _, n, d = parts.shape
    blk = 1000
    assert n % blk == 0

    def body(p_ref, wmu_ref, wrho_ref, ew_ref, bmu_ref, brho_ref, eb_ref, o_ref):
        w = wmu_ref[...] + jnp.log1p(jnp.exp(wrho_ref[...])) * ew_ref[...]
        b = bmu_ref[...] + jnp.log1p(jnp.exp(brho_ref[...])) * eb_ref[...]
        a = p_ref[0] + p_ref[1]
        y = jnp.dot(a, w, preferred_element_type=jnp.float32) + b
        o_ref[...] = jnp.maximum(y, 0.0) if relu else y

    full = pl.BlockSpec((d, d), lambda i: (0, 0))
    vec = pl.BlockSpec((1, d), lambda i: (0, 0))
    return pl.pallas_call(
        body,
        grid=(n // blk,),
        in_specs=[
            pl.BlockSpec((2, blk, d), lambda i: (0, i, 0)),
            full, full, full, vec, vec, vec,
        ],
        out_specs=pl.BlockSpec((blk, d), lambda i: (i, 0)),
        out_shape=jax.ShapeDtypeStruct((n, d), jnp.float32),
    )(parts, w_mu, w_rho, eps_w,
      b_mu.reshape(1, d), b_rho.reshape(1, d), eps_b.reshape(1, d))


def kernel(x, edge_index, W1_mu, W1_rho, b1_mu, b1_rho, W2_mu, W2_rho, b2_mu, b2_rho):
    k = jax.random.key(42)
    k1, k2 = jax.random.split(k)
    kW1, kb1 = jax.random.split(k1)
    kW2, kb2 = jax.random.split(k2)
    eps_W1 = jax.random.normal(kW1, W1_mu.shape, W1_mu.dtype)
    eps_b1 = jax.random.normal(kb1, b1_mu.shape, b1_mu.dtype)
    eps_W2 = jax.random.normal(kW2, W2_mu.shape, W2_mu.dtype)
    eps_b2 = jax.random.normal(kb2, b2_mu.shape, b2_mu.dtype)

    n = x.shape[0]
    src3, dst3 = _chunk_edges(edge_index[0], edge_index[1], n)

    p1 = _segment_sum_sc(x, src3, dst3)
    h = _dense_tc(p1, W1_mu, W1_rho, eps_W1, b1_mu, b1_rho, eps_b1, relu=True)
    p2 = _segment_sum_sc(h, src3, dst3)
    return _dense_tc(p2, W2_mu, W2_rho, eps_W2, b2_mu, b2_rho, eps_b2, relu=False)


# speedup vs baseline: 2.9080x; 2.9080x over previous
import functools

import jax
import jax.numpy as jnp
from jax import lax
from jax.experimental import pallas as pl
from jax.experimental.pallas import tpu as pltpu
from jax.experimental.pallas import tpu_sc as plsc

NC = 2
NS = 16
NW = NC * NS
CHUNK = 128


def _segment_sum_sc(table, src, dst):
    n, d = table.shape
    e = src.shape[0]
    assert e % CHUNK == 0
    n_chunks = e // CHUNK
    n_pairs, rem = divmod(n_chunks, 2 * NW)
    piece = 128
    n_full, tail = divmod(n, piece)
    assert tail % 8 == 0

    mesh = plsc.VectorSubcoreMesh(
        core_axis_name="c", subcore_axis_name="s", num_cores=NC, num_subcores=NS
    )

    @functools.partial(
        pl.kernel,
        out_type=jax.ShapeDtypeStruct((NC, n, d), jnp.float32),
        mesh=mesh,
        scratch_types=[
            [pltpu.VMEM((CHUNK,), jnp.int32) for _ in range(2)],
            [pltpu.VMEM((CHUNK,), jnp.int32) for _ in range(2)],
            [pltpu.VMEM((CHUNK, d), jnp.float32) for _ in range(2)],
            pltpu.VMEM_SHARED((n, d), jnp.float32),
            [pltpu.SemaphoreType.DMA for _ in range(2)],
            [pltpu.SemaphoreType.DMA for _ in range(2)],
            [pltpu.SemaphoreType.DMA for _ in range(2)],
        ],
    )
    def segsum(table_hbm, src_hbm, dst_hbm, out_hbm,
               src_v, dst_v, rows_v, acc_sh, semi, semg, sems):
        c = lax.axis_index("c")
        s = lax.axis_index("s")
        w = c * NS + s

        def zbody(i, carry):
            r = i // (d // 16)
            col = (i % (d // 16)) * 16
            rows_v[0][r, pl.ds(col, 16)] = jnp.zeros((16,), jnp.float32)
            return carry

        lax.fori_loop(0, piece * (d // 16), zbody, 0)

        my_pieces = (n_full - 1 - s) // NS + 1

        def zcopy(i, carry):
            r0 = (s + i * NS) * piece
            pltpu.sync_copy(rows_v[0].at[pl.ds(0, piece)], acc_sh.at[pl.ds(r0, piece)])
            return carry

        lax.fori_loop(0, my_pieces, zcopy, 0)
        if tail:
            @pl.when(s == NS - 1)
            def _():
                pltpu.sync_copy(rows_v[0].at[pl.ds(0, tail)],
                                acc_sh.at[pl.ds(n_full * piece, tail)])
        plsc.subcore_barrier()

        def ebody(t, carry):
            base0 = (w + (2 * t) * NW) * CHUNK
            base1 = (w + (2 * t + 1) * NW) * CHUNK
            ia = pltpu.async_copy(src_hbm.at[pl.ds(base0, CHUNK)], src_v[0], semi[0])
            ib = pltpu.async_copy(dst_hbm.at[pl.ds(base0, CHUNK)], dst_v[0], semi[0])
            ic = pltpu.async_copy(src_hbm.at[pl.ds(base1, CHUNK)], src_v[1], semi[1])
            id_ = pltpu.async_copy(dst_hbm.at[pl.ds(base1, CHUNK)], dst_v[1], semi[1])
            ia.wait()
            ib.wait()
            g0 = pltpu.async_copy(table_hbm.at[src_v[0]], rows_v[0], semg[0])
            ic.wait()
            id_.wait()
            g1 = pltpu.async_copy(table_hbm.at[src_v[1]], rows_v[1], semg[1])
            g0.wait()
            s0 = pltpu.async_copy(rows_v[0], acc_sh.at[dst_v[0]], sems[0], add=True)
            g1.wait()
            s1 = pltpu.async_copy(rows_v[1], acc_sh.at[dst_v[1]], sems[1], add=True)
            s0.wait()
            s1.wait()
            return carry

        lax.fori_loop(0, n_pairs, ebody, 0)

        if rem:
            @pl.when(w < rem)
            def _():
                base = (2 * n_pairs * NW + w) * CHUNK
                pltpu.sync_copy(src_hbm.at[pl.ds(base, CHUNK)], src_v[0])
                pltpu.sync_copy(dst_hbm.at[pl.ds(base, CHUNK)], dst_v[0])
                pltpu.async_copy(table_hbm.at[src_v[0]], rows_v[0], semg[0]).wait()
                pltpu.sync_copy(rows_v[0], acc_sh.at[dst_v[0]], add=True)
        plsc.subcore_barrier()

        def obody(i, carry):
            r0 = (s + i * NS) * piece
            pltpu.sync_copy(acc_sh.at[pl.ds(r0, piece)], rows_v[0].at[pl.ds(0, piece)])
            pltpu.sync_copy(rows_v[0].at[pl.ds(0, piece)], out_hbm.at[c].at[pl.ds(r0, piece)])
            return carry

        lax.fori_loop(0, my_pieces, obody, 0)
        if tail:
            @pl.when(s == NS - 1)
            def _():
                r0 = n_full * piece
                pltpu.sync_copy(acc_sh.at[pl.ds(r0, tail)], rows_v[0].at[pl.ds(0, tail)])
                pltpu.sync_copy(rows_v[0].at[pl.ds(0, tail)], out_hbm.at[c].at[pl.ds(r0, tail)])

    return segsum(table, src, dst)


def _dense_tc(parts, w_mu, w_rho, eps_w, b_mu, b_rho, eps_b, relu):
    _, n, d = parts.shape
    blk = 1000
    assert n % blk == 0

    def body(p_ref, wmu_ref, wrho_ref, ew_ref, bmu_ref, brho_ref, eb_ref, o_ref):
        w = wmu_ref[...] + jnp.log1p(jnp.exp(wrho_ref[...])) * ew_ref[...]
        b = bmu_ref[...] + jnp.log1p(jnp.exp(brho_ref[...])) * eb_ref[...]
        a = p_ref[0] + p_ref[1]
        y = jnp.dot(a, w, preferred_element_type=jnp.float32) + b
        o_ref[...] = jnp.maximum(y, 0.0) if relu else y

    full = pl.BlockSpec((d, d), lambda i: (0, 0))
    vec = pl.BlockSpec((1, d), lambda i: (0, 0))
    return pl.pallas_call(
        body,
        grid=(n // blk,),
        in_specs=[
            pl.BlockSpec((2, blk, d), lambda i: (0, i, 0)),
            full, full, full, vec, vec, vec,
        ],
        out_specs=pl.BlockSpec((blk, d), lambda i: (i, 0)),
        out_shape=jax.ShapeDtypeStruct((n, d), jnp.float32),
    )(parts, w_mu, w_rho, eps_w,
      b_mu.reshape(1, d), b_rho.reshape(1, d), eps_b.reshape(1, d))


def kernel(x, edge_index, W1_mu, W1_rho, b1_mu, b1_rho, W2_mu, W2_rho, b2_mu, b2_rho):
    k = jax.random.key(42)
    k1, k2 = jax.random.split(k)
    kW1, kb1 = jax.random.split(k1)
    kW2, kb2 = jax.random.split(k2)
    eps_W1 = jax.random.normal(kW1, W1_mu.shape, W1_mu.dtype)
    eps_b1 = jax.random.normal(kb1, b1_mu.shape, b1_mu.dtype)
    eps_W2 = jax.random.normal(kW2, W2_mu.shape, W2_mu.dtype)
    eps_b2 = jax.random.normal(kb2, b2_mu.shape, b2_mu.dtype)

    src = edge_index[0]
    dst = edge_index[1]

    p1 = _segment_sum_sc(x, src, dst)
    h = _dense_tc(p1, W1_mu, W1_rho, eps_W1, b1_mu, b1_rho, eps_b1, relu=True)
    p2 = _segment_sum_sc(h, src, dst)
    return _dense_tc(p2, W2_mu, W2_rho, eps_W2, b2_mu, b2_rho, eps_b2, relu=False)


# speedup vs baseline: 3.1009x; 1.0663x over previous
import functools

import jax
import jax.numpy as jnp
from jax import lax
from jax.experimental import pallas as pl
from jax.experimental.pallas import tpu as pltpu
from jax.experimental.pallas import tpu_sc as plsc

NC = 2
NS = 16
NW = NC * NS
CHUNK = 128
K = 3


def _segment_sum_sc(table, src, dst):
    n, d = table.shape
    e = src.shape[0]
    assert e % CHUNK == 0
    n_chunks = e // CHUNK
    n_iters, rem = divmod(n_chunks, K * NW)
    tail_iters = -(-rem // NW)
    piece = 128
    n_full, tail = divmod(n, piece)
    assert tail % 8 == 0

    mesh = plsc.VectorSubcoreMesh(
        core_axis_name="c", subcore_axis_name="s", num_cores=NC, num_subcores=NS
    )

    @functools.partial(
        pl.kernel,
        out_type=jax.ShapeDtypeStruct((NC, n, d), jnp.float32),
        mesh=mesh,
        scratch_types=[
            [pltpu.VMEM((CHUNK,), jnp.int32) for _ in range(K)],
            [pltpu.VMEM((CHUNK,), jnp.int32) for _ in range(K)],
            [pltpu.VMEM((CHUNK, d), jnp.float32) for _ in range(K)],
            pltpu.VMEM_SHARED((n, d), jnp.float32),
            [pltpu.SemaphoreType.DMA for _ in range(K)],
            [pltpu.SemaphoreType.DMA for _ in range(K)],
            [pltpu.SemaphoreType.DMA for _ in range(K)],
        ],
    )
    def segsum(table_hbm, src_hbm, dst_hbm, out_hbm,
               src_v, dst_v, rows_v, acc_sh, semi, semg, sems):
        c = lax.axis_index("c")
        s = lax.axis_index("s")
        w = c * NS + s

        def zbody(i, carry):
            r = i // (d // 16)
            col = (i % (d // 16)) * 16
            rows_v[0][r, pl.ds(col, 16)] = jnp.zeros((16,), jnp.float32)
            return carry

        lax.fori_loop(0, piece * (d // 16), zbody, 0)

        my_pieces = (n_full - 1 - s) // NS + 1

        def zcopy(i, carry):
            r0 = (s + i * NS) * piece
            pltpu.sync_copy(rows_v[0].at[pl.ds(0, piece)], acc_sh.at[pl.ds(r0, piece)])
            return carry

        lax.fori_loop(0, my_pieces, zcopy, 0)
        if tail:
            @pl.when(s == NS - 1)
            def _():
                pltpu.sync_copy(rows_v[0].at[pl.ds(0, tail)],
                                acc_sh.at[pl.ds(n_full * piece, tail)])
        plsc.subcore_barrier()

        def ebody(t, carry):
            idxd = []
            for q in range(K):
                base = (w + (K * t + q) * NW) * CHUNK
                idxd.append((
                    pltpu.async_copy(src_hbm.at[pl.ds(base, CHUNK)], src_v[q], semi[q]),
                    pltpu.async_copy(dst_hbm.at[pl.ds(base, CHUNK)], dst_v[q], semi[q]),
                ))
            gd = []
            for q in range(K):
                idxd[q][0].wait()
                idxd[q][1].wait()
                gd.append(pltpu.async_copy(table_hbm.at[src_v[q]], rows_v[q], semg[q]))
            sd = []
            for q in range(K):
                gd[q].wait()
                sd.append(pltpu.async_copy(rows_v[q], acc_sh.at[dst_v[q]],
                                           sems[q], add=True))
            for q in range(K):
                sd[q].wait()
            return carry

        lax.fori_loop(0, n_iters, ebody, 0)

        for q in range(tail_iters):
            @pl.when(w + q * NW < rem)
            def _():
                base = (K * n_iters * NW + q * NW + w) * CHUNK
                pltpu.sync_copy(src_hbm.at[pl.ds(base, CHUNK)], src_v[0])
                pltpu.sync_copy(dst_hbm.at[pl.ds(base, CHUNK)], dst_v[0])
                pltpu.async_copy(table_hbm.at[src_v[0]], rows_v[0], semg[0]).wait()
                pltpu.sync_copy(rows_v[0], acc_sh.at[dst_v[0]], add=True)
        plsc.subcore_barrier()

        def obody(i, carry):
            r0 = (s + i * NS) * piece
            pltpu.sync_copy(acc_sh.at[pl.ds(r0, piece)], rows_v[0].at[pl.ds(0, piece)])
            pltpu.sync_copy(rows_v[0].at[pl.ds(0, piece)], out_hbm.at[c].at[pl.ds(r0, piece)])
            return carry

        lax.fori_loop(0, my_pieces, obody, 0)
        if tail:
            @pl.when(s == NS - 1)
            def _():
                r0 = n_full * piece
                pltpu.sync_copy(acc_sh.at[pl.ds(r0, tail)], rows_v[0].at[pl.ds(0, tail)])
                pltpu.sync_copy(rows_v[0].at[pl.ds(0, tail)], out_hbm.at[c].at[pl.ds(r0, tail)])

    return segsum(table, src, dst)


def _dense_tc(parts, w_mu, w_rho, eps_w, b_mu, b_rho, eps_b, relu):
    _, n, d = parts.shape
    blk = 1000
    assert n % blk == 0

    def body(p_ref, wmu_ref, wrho_ref, ew_ref, bmu_ref, brho_ref, eb_ref, o_ref):
        w = wmu_ref[...] + jnp.log1p(jnp.exp(wrho_ref[...])) * ew_ref[...]
        b = bmu_ref[...] + jnp.log1p(jnp.exp(brho_ref[...])) * eb_ref[...]
        a = p_ref[0] + p_ref[1]
        y = jnp.dot(a, w, preferred_element_type=jnp.float32) + b
        o_ref[...] = jnp.maximum(y, 0.0) if relu else y

    full = pl.BlockSpec((d, d), lambda i: (0, 0))
    vec = pl.BlockSpec((1, d), lambda i: (0, 0))
    return pl.pallas_call(
        body,
        grid=(n // blk,),
        in_specs=[
            pl.BlockSpec((2, blk, d), lambda i: (0, i, 0)),
            full, full, full, vec, vec, vec,
        ],
        out_specs=pl.BlockSpec((blk, d), lambda i: (i, 0)),
        out_shape=jax.ShapeDtypeStruct((n, d), jnp.float32),
    )(parts, w_mu, w_rho, eps_w,
      b_mu.reshape(1, d), b_rho.reshape(1, d), eps_b.reshape(1, d))


def kernel(x, edge_index, W1_mu, W1_rho, b1_mu, b1_rho, W2_mu, W2_rho, b2_mu, b2_rho):
    k = jax.random.key(42)
    k1, k2 = jax.random.split(k)
    kW1, kb1 = jax.random.split(k1)
    kW2, kb2 = jax.random.split(k2)
    eps_W1 = jax.random.normal(kW1, W1_mu.shape, W1_mu.dtype)
    eps_b1 = jax.random.normal(kb1, b1_mu.shape, b1_mu.dtype)
    eps_W2 = jax.random.normal(kW2, W2_mu.shape, W2_mu.dtype)
    eps_b2 = jax.random.normal(kb2, b2_mu.shape, b2_mu.dtype)

    src = edge_index[0]
    dst = edge_index[1]

    p1 = _segment_sum_sc(x, src, dst)
    h = _dense_tc(p1, W1_mu, W1_rho, eps_W1, b1_mu, b1_rho, eps_b1, relu=True)
    p2 = _segment_sum_sc(h, src, dst)
    return _dense_tc(p2, W2_mu, W2_rho, eps_W2, b2_mu, b2_rho, eps_b2, relu=False)


# speedup vs baseline: 3.1034x; 1.0008x over previous
import functools

import jax
import jax.numpy as jnp
from jax import lax
from jax.experimental import pallas as pl
from jax.experimental.pallas import tpu as pltpu
from jax.experimental.pallas import tpu_sc as plsc

NC = 2
NS = 16
NW = NC * NS
CHUNK = 128
K = 3


def _segment_sum_sc(table, src, dst):
    n, d = table.shape
    e = src.shape[0]
    assert e % CHUNK == 0
    n_chunks = e // CHUNK
    n_iters, rem = divmod(n_chunks, K * NW)
    tail_iters = -(-rem // NW)
    piece = 128
    n_full, tail = divmod(n, piece)
    assert tail % 8 == 0

    mesh = plsc.VectorSubcoreMesh(
        core_axis_name="c", subcore_axis_name="s", num_cores=NC, num_subcores=NS
    )

    @functools.partial(
        pl.kernel,
        out_type=jax.ShapeDtypeStruct((NC, n, d), jnp.float32),
        mesh=mesh,
        scratch_types=[
            [pltpu.VMEM((CHUNK,), jnp.int32) for _ in range(K)],
            [pltpu.VMEM((CHUNK,), jnp.int32) for _ in range(K)],
            [pltpu.VMEM((CHUNK, d), jnp.float32) for _ in range(K)],
            pltpu.VMEM_SHARED((n, d), jnp.float32),
            [pltpu.SemaphoreType.DMA for _ in range(K)],
            [pltpu.SemaphoreType.DMA for _ in range(K)],
            [pltpu.SemaphoreType.DMA for _ in range(K)],
        ],
    )
    def segsum(table_hbm, src_hbm, dst_hbm, out_hbm,
               src_v, dst_v, rows_v, acc_sh, semi, semg, sems):
        c = lax.axis_index("c")
        s = lax.axis_index("s")
        w = c * NS + s

        def zbody(i, carry):
            r = i // (d // 16)
            col = (i % (d // 16)) * 16
            rows_v[0][r, pl.ds(col, 16)] = jnp.zeros((16,), jnp.float32)
            return carry

        lax.fori_loop(0, piece * (d // 16), zbody, 0)

        my_pieces = (n_full - 1 - s) // NS + 1

        def zcopy(i, carry):
            r0 = (s + i * NS) * piece
            pltpu.async_copy(rows_v[0].at[pl.ds(0, piece)],
                             acc_sh.at[pl.ds(r0, piece)], semi[0])
            return carry

        lax.fori_loop(0, my_pieces, zcopy, 0)

        def zwait(i, carry):
            pltpu.make_async_copy(rows_v[0].at[pl.ds(0, piece)],
                                  acc_sh.at[pl.ds(0, piece)], semi[0]).wait()
            return carry

        lax.fori_loop(0, my_pieces, zwait, 0)
        if tail:
            @pl.when(s == NS - 1)
            def _():
                pltpu.sync_copy(rows_v[0].at[pl.ds(0, tail)],
                                acc_sh.at[pl.ds(n_full * piece, tail)])
        plsc.subcore_barrier()

        def ebody(t, carry):
            idxd = []
            for q in range(K):
                base = (w + (K * t + q) * NW) * CHUNK
                idxd.append((
                    pltpu.async_copy(src_hbm.at[pl.ds(base, CHUNK)], src_v[q], semi[q]),
                    pltpu.async_copy(dst_hbm.at[pl.ds(base, CHUNK)], dst_v[q], semi[q]),
                ))
            gd = []
            for q in range(K):
                idxd[q][0].wait()
                idxd[q][1].wait()
                gd.append(pltpu.async_copy(table_hbm.at[src_v[q]], rows_v[q], semg[q]))
            sd = []
            for q in range(K):
                gd[q].wait()
                sd.append(pltpu.async_copy(rows_v[q], acc_sh.at[dst_v[q]],
                                           sems[q], add=True))
            for q in range(K):
                sd[q].wait()
            return carry

        lax.fori_loop(0, n_iters, ebody, 0)

        for q in range(tail_iters):
            @pl.when(w + q * NW < rem)
            def _():
                base = (K * n_iters * NW + q * NW + w) * CHUNK
                pltpu.sync_copy(src_hbm.at[pl.ds(base, CHUNK)], src_v[0])
                pltpu.sync_copy(dst_hbm.at[pl.ds(base, CHUNK)], dst_v[0])
                pltpu.async_copy(table_hbm.at[src_v[0]], rows_v[0], semg[0]).wait()
                pltpu.sync_copy(rows_v[0], acc_sh.at[dst_v[0]], add=True)
        plsc.subcore_barrier()

        def obody(i, carry):
            r0 = (s + i * NS) * piece
            pltpu.async_copy(acc_sh.at[pl.ds(r0, piece)],
                             out_hbm.at[c].at[pl.ds(r0, piece)], sems[0])
            return carry

        lax.fori_loop(0, my_pieces, obody, 0)

        def owait(i, carry):
            pltpu.make_async_copy(acc_sh.at[pl.ds(0, piece)],
                                  out_hbm.at[c].at[pl.ds(0, piece)], sems[0]).wait()
            return carry

        lax.fori_loop(0, my_pieces, owait, 0)
        if tail:
            @pl.when(s == NS - 1)
            def _():
                r0 = n_full * piece
                pltpu.sync_copy(acc_sh.at[pl.ds(r0, tail)],
                                out_hbm.at[c].at[pl.ds(r0, tail)])

    return segsum(table, src, dst)


def _dense_tc(parts, w_mu, w_rho, eps_w, b_mu, b_rho, eps_b, relu):
    _, n, d = parts.shape
    blk = 1000
    assert n % blk == 0

    def body(p_ref, wmu_ref, wrho_ref, ew_ref, bmu_ref, brho_ref, eb_ref, o_ref):
        w = wmu_ref[...] + jnp.log1p(jnp.exp(wrho_ref[...])) * ew_ref[...]
        b = bmu_ref[...] + jnp.log1p(jnp.exp(brho_ref[...])) * eb_ref[...]
        a = p_ref[0] + p_ref[1]
        y = jnp.dot(a, w, preferred_element_type=jnp.float32) + b
        o_ref[...] = jnp.maximum(y, 0.0) if relu else y

    full = pl.BlockSpec((d, d), lambda i: (0, 0))
    vec = pl.BlockSpec((1, d), lambda i: (0, 0))
    return pl.pallas_call(
        body,
        grid=(n // blk,),
        in_specs=[
            pl.BlockSpec((2, blk, d), lambda i: (0, i, 0)),
            full, full, full, vec, vec, vec,
        ],
        out_specs=pl.BlockSpec((blk, d), lambda i: (i, 0)),
        out_shape=jax.ShapeDtypeStruct((n, d), jnp.float32),
    )(parts, w_mu, w_rho, eps_w,
      b_mu.reshape(1, d), b_rho.reshape(1, d), eps_b.reshape(1, d))


def kernel(x, edge_index, W1_mu, W1_rho, b1_mu, b1_rho, W2_mu, W2_rho, b2_mu, b2_rho):
    k = jax.random.key(42)
    k1, k2 = jax.random.split(k)
    kW1, kb1 = jax.random.split(k1)
    kW2, kb2 = jax.random.split(k2)
    eps_W1 = jax.random.normal(kW1, W1_mu.shape, W1_mu.dtype)
    eps_b1 = jax.random.normal(kb1, b1_mu.shape, b1_mu.dtype)
    eps_W2 = jax.random.normal(kW2, W2_mu.shape, W2_mu.dtype)
    eps_b2 = jax.random.normal(kb2, b2_mu.shape, b2_mu.dtype)

    src = edge_index[0]
    dst = edge_index[1]

    p1 = _segment_sum_sc(x, src, dst)
    h = _dense_tc(p1, W1_mu, W1_rho, eps_W1, b1_mu, b1_rho, eps_b1, relu=True)
    p2 = _segment_sum_sc(h, src, dst)
    return _dense_tc(p2, W2_mu, W2_rho, eps_W2, b2_mu, b2_rho, eps_b2, relu=False)
